# baseline (device time: 13404 ns/iter reference)
import jax
import jax.numpy as jnp
from jax import lax
from jax.experimental import pallas as pl
from jax.experimental.pallas import tpu as pltpu

MSUB = 2
DSUB = 4
C = MSUB * DSUB
BLK = 128


def kernel(dy, W):
    dy = pltpu.with_memory_space_constraint(dy, pltpu.MemorySpace.HBM)
    W = pltpu.with_memory_space_constraint(W, pltpu.MemorySpace.HBM)
    m, k = dy.shape
    d = W.shape[0]
    mh = m // 2

    def body(dy_ref, w_ref, out_ref,
             dyv, wv, ysend, yrecv, outv,
             load_sems, ysend_sems, yrecv_sems, xsend_sems, xrecv_sems,
             out_sems):
        my_x = lax.axis_index("x")
        my_y = lax.axis_index("y")
        my_z = lax.axis_index("z")

        barrier = pltpu.get_barrier_semaphore()
        pl.semaphore_signal(
            barrier, inc=1, device_id=(my_x, 1 - my_y, my_z),
            device_id_type=pl.DeviceIdType.MESH)
        pl.semaphore_signal(
            barrier, inc=1, device_id=(1 - my_x, my_y, my_z),
            device_id_type=pl.DeviceIdType.MESH)

        dy_dmas = []
        for mi in range(MSUB):
            dd = pltpu.make_async_copy(
                dy_ref.at[pl.ds(my_x * mh + mi * BLK, BLK), :],
                dyv.at[pl.ds(mi * BLK, BLK), :],
                load_sems.at[mi])
            dd.start()
            dy_dmas.append(dd)
        w0_dma = pltpu.make_async_copy(
            w_ref.at[pl.ds(0, BLK), :], wv.at[pl.ds(0, BLK), :],
            load_sems.at[MSUB])
        w0_dma.start()
        w1_dma = pltpu.make_async_copy(
            w_ref.at[pl.ds(BLK, d - BLK), :], wv.at[pl.ds(BLK, d - BLK), :],
            load_sems.at[MSUB + 1])
        w1_dma.start()

        dyb = [None] * MSUB
        y_rdmas = []
        for c in range(C):
            dc, mi = divmod(c, MSUB)
            if c == 0:
                w0_dma.wait()
            elif c == MSUB:
                w1_dma.wait()
            if dc == 0:
                dy_dmas[mi].wait()
                dyb[mi] = dyv[pl.ds(mi * BLK, BLK), :].astype(jnp.bfloat16)
            part_c = lax.dot_general(
                dyb[mi],
                wv[pl.ds(dc * BLK, BLK), :].astype(jnp.bfloat16),
                dimension_numbers=(((1,), (1,)), ((), ())),
                preferred_element_type=jnp.float32,
            )
            ysend[c] = part_c.astype(jnp.bfloat16)
            if c == 0:
                pl.semaphore_wait(barrier, 2)
            yr = pltpu.make_async_remote_copy(
                src_ref=ysend.at[c], dst_ref=yrecv.at[c],
                send_sem=ysend_sems.at[c], recv_sem=yrecv_sems.at[c],
                device_id=(my_x, 1 - my_y, my_z),
                device_id_type=pl.DeviceIdType.MESH,
            )
            yr.start()
            y_rdmas.append(yr)

        x_rdmas = []
        for c in range(C):
            dc, mi = divmod(c, MSUB)
            rows = pl.ds(my_x * mh + mi * BLK, BLK)
            cols = pl.ds(dc * BLK, BLK)
            y_rdmas[c].wait()
            outv[rows, cols] = ysend[c] + yrecv[c]
            xr = pltpu.make_async_remote_copy(
                src_ref=outv.at[rows, cols], dst_ref=outv.at[rows, cols],
                send_sem=xsend_sems.at[c], recv_sem=xrecv_sems.at[c],
                device_id=(1 - my_x, my_y, my_z),
                device_id_type=pl.DeviceIdType.MESH,
            )
            xr.start()
            x_rdmas.append(xr)

        out_my = pltpu.make_async_copy(
            outv.at[pl.ds(my_x * mh, mh), :],
            out_ref.at[pl.ds(my_x * mh, mh), :],
            out_sems.at[0])
        out_my.start()

        out_others = []
        for c in range(C):
            dc, mi = divmod(c, MSUB)
            rows = pl.ds((1 - my_x) * mh + mi * BLK, BLK)
            cols = pl.ds(dc * BLK, BLK)
            x_rdmas[c].wait()
            oc = pltpu.make_async_copy(
                outv.at[rows, cols], out_ref.at[rows, cols],
                out_sems.at[1 + c])
            oc.start()
            out_others.append(oc)

        out_my.wait()
        for oc in out_others:
            oc.wait()

    return pl.pallas_call(
        body,
        out_shape=jax.ShapeDtypeStruct((m, d), jnp.bfloat16),
        in_specs=[
            pl.BlockSpec(memory_space=pltpu.MemorySpace.HBM),
            pl.BlockSpec(memory_space=pltpu.MemorySpace.HBM),
        ],
        out_specs=pl.BlockSpec(memory_space=pltpu.MemorySpace.HBM),
        scratch_shapes=[
            pltpu.VMEM((mh, k), jnp.float32),
            pltpu.VMEM((d, k), jnp.float32),
            pltpu.VMEM((C, BLK, BLK), jnp.bfloat16),
            pltpu.VMEM((C, BLK, BLK), jnp.bfloat16),
            pltpu.VMEM((m, d), jnp.bfloat16),
            pltpu.SemaphoreType.DMA((MSUB + 2,)),
            pltpu.SemaphoreType.DMA((C,)),
            pltpu.SemaphoreType.DMA((C,)),
            pltpu.SemaphoreType.DMA((C,)),
            pltpu.SemaphoreType.DMA((C,)),
            pltpu.SemaphoreType.DMA((C + 1,)),
        ],
        compiler_params=pltpu.CompilerParams(collective_id=0),
    )(dy, W)


# device time: 13283 ns/iter; 1.0091x vs baseline; 1.0091x over previous
import jax
import jax.numpy as jnp
from jax import lax
from jax.experimental import pallas as pl
from jax.experimental.pallas import tpu as pltpu

C = 4


def kernel(dy, W):
    dy = pltpu.with_memory_space_constraint(dy, pltpu.MemorySpace.HBM)
    W = pltpu.with_memory_space_constraint(W, pltpu.MemorySpace.HBM)
    m, k = dy.shape
    d = W.shape[0]
    mh = m // 2
    dc = d // C

    def body(dy_ref, w_ref, out_ref,
             dyv, wv, ysend, yrecv, outv,
             load_sems, ysend_sems, yrecv_sems, xsend_sems, xrecv_sems,
             out_sems):
        my_x = lax.axis_index("x")
        my_y = lax.axis_index("y")
        my_z = lax.axis_index("z")

        barrier = pltpu.get_barrier_semaphore()
        pl.semaphore_signal(
            barrier, inc=1, device_id=(my_x, 1 - my_y, my_z),
            device_id_type=pl.DeviceIdType.MESH)
        pl.semaphore_signal(
            barrier, inc=1, device_id=(1 - my_x, my_y, my_z),
            device_id_type=pl.DeviceIdType.MESH)

        dy_dma = pltpu.make_async_copy(
            dy_ref.at[pl.ds(my_x * mh, mh), :], dyv, load_sems.at[0])
        dy_dma.start()
        w0_dma = pltpu.make_async_copy(
            w_ref.at[pl.ds(0, dc), :], wv.at[pl.ds(0, dc), :],
            load_sems.at[1])
        w0_dma.start()
        w1_dma = pltpu.make_async_copy(
            w_ref.at[pl.ds(dc, d - dc), :], wv.at[pl.ds(dc, d - dc), :],
            load_sems.at[2])
        w1_dma.start()

        dy_dma.wait()
        dyb = dyv[...].astype(jnp.bfloat16)

        y_rdmas = []
        for c in range(C):
            if c == 0:
                w0_dma.wait()
            elif c == 1:
                w1_dma.wait()
            part_c = lax.dot_general(
                dyb,
                wv[pl.ds(c * dc, dc), :].astype(jnp.bfloat16),
                dimension_numbers=(((1,), (1,)), ((), ())),
                preferred_element_type=jnp.float32,
            )
            ysend[c] = part_c.astype(jnp.bfloat16)
            if c == 0:
                pl.semaphore_wait(barrier, 2)
            yr = pltpu.make_async_remote_copy(
                src_ref=ysend.at[c], dst_ref=yrecv.at[c],
                send_sem=ysend_sems.at[c], recv_sem=yrecv_sems.at[c],
                device_id=(my_x, 1 - my_y, my_z),
                device_id_type=pl.DeviceIdType.MESH,
            )
            yr.start()
            y_rdmas.append(yr)

        x_rdmas = []
        for c in range(C):
            rows = pl.ds(my_x * mh, mh)
            cols = pl.ds(c * dc, dc)
            y_rdmas[c].wait()
            outv[rows, cols] = ysend[c] + yrecv[c]
            xr = pltpu.make_async_remote_copy(
                src_ref=outv.at[rows, cols], dst_ref=outv.at[rows, cols],
                send_sem=xsend_sems.at[c], recv_sem=xrecv_sems.at[c],
                device_id=(1 - my_x, my_y, my_z),
                device_id_type=pl.DeviceIdType.MESH,
            )
            xr.start()
            x_rdmas.append(xr)

        out_my = pltpu.make_async_copy(
            outv.at[pl.ds(my_x * mh, mh), :],
            out_ref.at[pl.ds(my_x * mh, mh), :],
            out_sems.at[0])
        out_my.start()

        out_others = []
        for c in range(C):
            rows = pl.ds((1 - my_x) * mh, mh)
            cols = pl.ds(c * dc, dc)
            x_rdmas[c].wait()
            oc = pltpu.make_async_copy(
                outv.at[rows, cols], out_ref.at[rows, cols],
                out_sems.at[1 + c])
            oc.start()
            out_others.append(oc)

        out_my.wait()
        for oc in out_others:
            oc.wait()

    return pl.pallas_call(
        body,
        out_shape=jax.ShapeDtypeStruct((m, d), jnp.bfloat16),
        in_specs=[
            pl.BlockSpec(memory_space=pltpu.MemorySpace.HBM),
            pl.BlockSpec(memory_space=pltpu.MemorySpace.HBM),
        ],
        out_specs=pl.BlockSpec(memory_space=pltpu.MemorySpace.HBM),
        scratch_shapes=[
            pltpu.VMEM((mh, k), jnp.float32),
            pltpu.VMEM((d, k), jnp.float32),
            pltpu.VMEM((C, mh, dc), jnp.bfloat16),
            pltpu.VMEM((C, mh, dc), jnp.bfloat16),
            pltpu.VMEM((m, d), jnp.bfloat16),
            pltpu.SemaphoreType.DMA((3,)),
            pltpu.SemaphoreType.DMA((C,)),
            pltpu.SemaphoreType.DMA((C,)),
            pltpu.SemaphoreType.DMA((C,)),
            pltpu.SemaphoreType.DMA((C,)),
            pltpu.SemaphoreType.DMA((C + 1,)),
        ],
        compiler_params=pltpu.CompilerParams(collective_id=0),
    )(dy, W)
